# baseline (speedup 1.0000x reference)
import functools

import jax
import jax.numpy as jnp
from jax import lax
from jax.experimental import pallas as pl
from jax.experimental.pallas import tpu as pltpu
from jax.experimental.pallas import tpu_sc as plsc

_B = 16384 * 200
_D = 32
_NC = 2
_NS = 16
_NW = _NC * _NS
_BPW = _B // _NW
_CHUNK = 512
_NIT = _BPW // _CHUNK

_SEQ = 200
_BATCH = 16384
_NBLK = _BATCH // _CHUNK
_TD = _D // 8
_TI = _CHUNK // 128

assert _BPW % _CHUNK == 0 and _NIT % 2 == 0


def _make_gather():
    mesh = plsc.VectorSubcoreMesh(core_axis_name="c", subcore_axis_name="s")

    scratch = (
        [pltpu.VMEM((_CHUNK,), jnp.int32) for _ in range(2)]
        + [pltpu.VMEM((_CHUNK, _D), jnp.float32) for _ in range(2)]
        + [pltpu.VMEM((_CHUNK * _D,), jnp.float32) for _ in range(2)]
        + [pltpu.VMEM((_D,), jnp.int32)]
        + [pltpu.SemaphoreType.DMA for _ in range(2)]
        + [pltpu.SemaphoreType.DMA for _ in range(2)]
        + [pltpu.SemaphoreType.DMA for _ in range(2)]
    )

    @functools.partial(
        pl.kernel,
        mesh=mesh,
        out_type=jax.ShapeDtypeStruct((_SEQ, _TD, 128 * 8 * 128),
                                      jnp.float32),
        scratch_types=scratch,
        compiler_params=pltpu.CompilerParams(use_tc_tiling_on_sc=False,
                                             needs_layout_passes=False),
    )
    def gather(idx_hbm, table_hbm, out_hbm, *refs):
        idx_bufs = refs[0:2]
        row_bufs = refs[2:4]
        tbs = refs[4:6]
        ftab = refs[6]
        isems = refs[7:9]
        gsems = refs[9:11]
        osems = refs[11:13]

        wid = lax.axis_index("s") * _NC + lax.axis_index("c")
        base = wid * _BPW
        lane = lax.iota(jnp.int32, 16)

        for q in range(_D // 16):
            c = lane + q * 16
            ftab[pl.ds(q * 16, 16)] = (
                (c >> 3) * (_TI * 8 * 128) + jnp.bitwise_and(c, 7) * 128)

        def off(it):
            return pl.multiple_of(base + it * _CHUNK, 32)

        def start_idx(it, b):
            pltpu.async_copy(idx_hbm.at[pl.ds(off(it), _CHUNK)], idx_bufs[b],
                             isems[b])

        def start_gather(b):
            pltpu.async_copy(table_hbm.at[idx_bufs[b]], row_bufs[b], gsems[b])

        def wait_isem(b):
            pltpu.make_async_copy(
                idx_hbm.at[pl.ds(0, _CHUNK)], idx_bufs[b], isems[b]).wait()

        def wait_gsem(b):
            pltpu.make_async_copy(
                table_hbm.at[idx_bufs[b]], row_bufs[b], gsems[b]).wait()

        def wait_osem(b):
            pltpu.make_async_copy(
                tbs[b], out_hbm.at[0, 0, pl.ds(0, _CHUNK * _D)],
                osems[b]).wait()

        def body(it, b, first, has_next, has_idx2):
            b1 = 1 - b
            wait_gsem(b)
            if has_next:
                wait_isem(b1)
                start_gather(b1)
            if has_idx2:
                start_idx(it + 2, b)
            if not first:
                wait_osem(b)
            def gloop(g, carry):
                ridx = lane + g * 16
                pos = ((ridx >> 7) << 10) + jnp.bitwise_and(ridx, 127)
                for d in range(_D):
                    rot = jnp.bitwise_and(lane + d, _D - 1)
                    fv = plsc.load_gather(ftab, [rot])
                    v = plsc.load_gather(row_bufs[b], [ridx, rot])
                    plsc.store_scatter(tbs[b], [pos + fv], v)
                return carry
            lax.fori_loop(0, _CHUNK // 16, gloop, 0)
            c = base // _CHUNK + it
            j = c // _NBLK
            x0 = pl.multiple_of((c % _NBLK) * (_TI * 1024), 1024)
            for td in range(_TD):
                pltpu.async_copy(
                    tbs[b].at[pl.ds(td * (_TI * 1024), _TI * 1024)],
                    out_hbm.at[j, td, pl.ds(x0, _TI * 1024)],
                    osems[b])

        start_idx(0, 0)
        start_idx(1, 1)
        wait_isem(0)
        start_gather(0)

        body(0, 0, first=True, has_next=True, has_idx2=True)
        body(1, 1, first=True, has_next=True, has_idx2=True)

        def pair(p, carry):
            it = p * 2
            body(it, 0, first=False, has_next=True, has_idx2=True)
            body(it + 1, 1, first=False, has_next=True, has_idx2=True)
            return carry

        lax.fori_loop(1, _NIT // 2 - 1, pair, 0)

        body(_NIT - 2, 0, first=False, has_next=True, has_idx2=False)
        body(_NIT - 1, 1, first=False, has_next=False, has_idx2=False)

        wait_osem(0)
        wait_osem(1)

    return gather


_gather = _make_gather()


def kernel(x, weight):
    idx = x.T.reshape(-1).astype(jnp.int32)
    o = _gather(idx, weight)
    o = o.reshape(_SEQ, _TD, 128, 8, 128)
    return jnp.transpose(o, (2, 4, 0, 1, 3)).reshape(_BATCH, _SEQ, _D)


# speedup vs baseline: 1.4159x; 1.4159x over previous
import functools

import jax
import jax.numpy as jnp
from jax import lax
from jax.experimental import pallas as pl
from jax.experimental.pallas import tpu as pltpu
from jax.experimental.pallas import tpu_sc as plsc

_B = 16384 * 200
_D = 32
_NC = 2
_NS = 16
_NW = _NC * _NS
_BPW = _B // _NW
_CHUNK = 512
_NIT = _BPW // _CHUNK

_SEQ = 200
_BATCH = 16384
_NBLK = _BATCH // _CHUNK
_TD = _D // 8
_TI = _CHUNK // 128

assert _BPW % _CHUNK == 0 and _NIT % 2 == 0


def _make_gather():
    mesh = plsc.VectorSubcoreMesh(core_axis_name="c", subcore_axis_name="s")

    scratch = (
        [pltpu.VMEM((_CHUNK,), jnp.int32) for _ in range(2)]
        + [pltpu.VMEM((_CHUNK, _D), jnp.float32) for _ in range(2)]
        + [pltpu.VMEM((_CHUNK * _D,), jnp.float32) for _ in range(2)]
        + [pltpu.SemaphoreType.DMA for _ in range(2)]
        + [pltpu.SemaphoreType.DMA for _ in range(2)]
        + [pltpu.SemaphoreType.DMA for _ in range(2)]
    )

    @functools.partial(
        pl.kernel,
        mesh=mesh,
        out_type=jax.ShapeDtypeStruct((_SEQ, _TD, 128 * 8 * 128),
                                      jnp.float32),
        scratch_types=scratch,
        compiler_params=pltpu.CompilerParams(use_tc_tiling_on_sc=False,
                                             needs_layout_passes=False),
    )
    def gather(idx_hbm, table_hbm, out_hbm, *refs):
        idx_bufs = refs[0:2]
        row_bufs = refs[2:4]
        tbs = refs[4:6]
        isems = refs[6:8]
        gsems = refs[8:10]
        osems = refs[10:12]

        wid = lax.axis_index("s") * _NC + lax.axis_index("c")
        base = wid * _BPW
        lane = lax.iota(jnp.int32, 16)

        def off(it):
            return pl.multiple_of(base + it * _CHUNK, 32)

        def start_idx(it, b):
            pltpu.async_copy(idx_hbm.at[pl.ds(off(it), _CHUNK)], idx_bufs[b],
                             isems[b])

        def start_gather(b):
            pltpu.async_copy(table_hbm.at[idx_bufs[b]], row_bufs[b], gsems[b])

        def wait_isem(b):
            pltpu.make_async_copy(
                idx_hbm.at[pl.ds(0, _CHUNK)], idx_bufs[b], isems[b]).wait()

        def wait_gsem(b):
            pltpu.make_async_copy(
                table_hbm.at[idx_bufs[b]], row_bufs[b], gsems[b]).wait()

        def wait_osem(b):
            pltpu.make_async_copy(
                tbs[b], out_hbm.at[0, 0, pl.ds(0, _CHUNK * _D)],
                osems[b]).wait()

        def body(it, b, first, has_next, has_idx2):
            b1 = 1 - b
            wait_gsem(b)
            if has_next:
                wait_isem(b1)
                start_gather(b1)
            if has_idx2:
                start_idx(it + 2, b)
            if not first:
                wait_osem(b)
            def dloop(d, carry):
                rot = jnp.bitwise_and(lane + d, _D - 1)
                fv = ((rot >> 3) << 12) + (jnp.bitwise_and(rot, 7) << 7)
                lfv = lane + fv
                for g in range(_CHUNK // 16):
                    ridx = lane + g * 16
                    v = plsc.load_gather(row_bufs[b], [ridx, rot])
                    plsc.store_scatter(
                        tbs[b],
                        [lfv + ((g // 8) * 1024 + (g % 8) * 16)], v)
                return carry
            lax.fori_loop(0, _D, dloop, 0)
            c = base // _CHUNK + it
            j = c // _NBLK
            x0 = pl.multiple_of((c % _NBLK) * (_TI * 1024), 1024)
            for td in range(_TD):
                pltpu.async_copy(
                    tbs[b].at[pl.ds(td * (_TI * 1024), _TI * 1024)],
                    out_hbm.at[j, td, pl.ds(x0, _TI * 1024)],
                    osems[b])

        start_idx(0, 0)
        start_idx(1, 1)
        wait_isem(0)
        start_gather(0)

        body(0, 0, first=True, has_next=True, has_idx2=True)
        body(1, 1, first=True, has_next=True, has_idx2=True)

        def pair(p, carry):
            it = p * 2
            body(it, 0, first=False, has_next=True, has_idx2=True)
            body(it + 1, 1, first=False, has_next=True, has_idx2=True)
            return carry

        lax.fori_loop(1, _NIT // 2 - 1, pair, 0)

        body(_NIT - 2, 0, first=False, has_next=True, has_idx2=False)
        body(_NIT - 1, 1, first=False, has_next=False, has_idx2=False)

        wait_osem(0)
        wait_osem(1)

    return gather


_gather = _make_gather()


def kernel(x, weight):
    idx = x.T.reshape(-1).astype(jnp.int32)
    o = _gather(idx, weight)
    o = o.reshape(_SEQ, _TD, 128, 8, 128)
    return jnp.transpose(o, (2, 4, 0, 1, 3)).reshape(_BATCH, _SEQ, _D)
